# baseline (speedup 1.0000x reference)
import functools
import math

import jax
import jax.numpy as jnp
import numpy as np
from jax import lax
from jax.experimental import pallas as pl
from jax.experimental.pallas import tpu as pltpu
from jax.experimental.pallas import tpu_sc as plsc

_NUM_CORES = 2
_NUM_SUBCORES = 16
_LANES = 16
_NW = _NUM_CORES * _NUM_SUBCORES
_NBUF = 4
_N_CHUNKS = 8


@functools.lru_cache(maxsize=None)
def _pos_encoding_np(seq_len, d_model):
    pos = np.arange(seq_len, dtype=np.float32)[:, None]
    div = np.exp(
        np.arange(0, d_model, 2, dtype=np.float32) * (-np.log(10000.0) / d_model)
    )
    pe = np.zeros((seq_len, d_model), dtype=np.float32)
    pe[:, 0::2] = np.sin(pos * div)
    pe[:, 1::2] = np.cos(pos * div)
    return pe


@functools.lru_cache(maxsize=None)
def _build_kernel(B, L, D):
    assert L % (_NW * _N_CHUNKS) == 0 and D % _LANES == 0
    rows_per_w = L // _NW
    cp = rows_per_w // _N_CHUNKS
    scale = math.sqrt(float(D))
    groups = D // _LANES

    mesh = plsc.VectorSubcoreMesh(core_axis_name="c", subcore_axis_name="s")

    @functools.partial(
        pl.kernel,
        mesh=mesh,
        out_type=jax.ShapeDtypeStruct((B, L, D), jnp.float32),
        scratch_types=[
            pltpu.VMEM((B, rows_per_w), jnp.int32),
            pltpu.VMEM((rows_per_w, D), jnp.float32),
        ]
        + [pltpu.VMEM((B * cp, D), jnp.float32) for _ in range(_NBUF)]
        + [pltpu.SemaphoreType.DMA for _ in range(2 * _NBUF)],
    )
    def emb_kernel(x_hbm, pe_hbm, table_hbm, out_hbm, idx_v, pe_v, *bufs_sems):
        bufs = bufs_sems[:_NBUF]
        gsems = bufs_sems[_NBUF : 2 * _NBUF]
        ssems = bufs_sems[2 * _NBUF :]

        wid = lax.axis_index("s") * _NUM_CORES + lax.axis_index("c")
        pos0 = wid * rows_per_w

        for b in range(B):
            pltpu.sync_copy(x_hbm.at[b, pl.ds(pos0, rows_per_w)], idx_v.at[b])
        pltpu.sync_copy(pe_hbm.at[pl.ds(pos0, rows_per_w)], pe_v)

        def start_gather(c):
            buf, sem = bufs[c % _NBUF], gsems[c % _NBUF]
            return [
                pltpu.async_copy(
                    table_hbm.at[idx_v.at[b, pl.ds(c * cp, cp)]],
                    buf.at[pl.ds(b * cp, cp)],
                    sem,
                )
                for b in range(B)
            ]

        def start_store(c):
            buf, sem = bufs[c % _NBUF], ssems[c % _NBUF]
            return [
                pltpu.async_copy(
                    buf.at[pl.ds(b * cp, cp)],
                    out_hbm.at[b, pl.ds(pos0 + c * cp, cp)],
                    sem,
                )
                for b in range(B)
            ]

        gcopies = {}
        scopies = {}
        for c in range(_NBUF - 1):
            gcopies[c] = start_gather(c)
        for c in range(_N_CHUNKS):
            for h in gcopies[c]:
                h.wait()
            buf = bufs[c % _NBUF]

            def row_body(r, carry, buf=buf, c=c):
                for j in range(groups):
                    sl = pl.ds(j * _LANES, _LANES)
                    p = pe_v[c * cp + r, sl]
                    for b in range(B):
                        buf[b * cp + r, sl] = buf[b * cp + r, sl] * scale + p
                return carry

            lax.fori_loop(0, cp, row_body, 0)

            scopies[c] = start_store(c)
            nxt = c + _NBUF - 1
            if nxt < _N_CHUNKS:
                prev = nxt - _NBUF
                if prev >= 0:
                    for h in scopies[prev]:
                        h.wait()
                gcopies[nxt] = start_gather(nxt)
        for c in range(max(0, _N_CHUNKS - _NBUF), _N_CHUNKS):
            for h in scopies[c]:
                h.wait()

    return emb_kernel


def kernel(x, table):
    B, L = x.shape
    V, D = table.shape
    pe = jnp.asarray(_pos_encoding_np(L, D))
    return _build_kernel(B, L, D)(x.astype(jnp.int32), pe, table)
